# Initial kernel scaffold; baseline (speedup 1.0000x reference)
import jax
import jax.numpy as jnp
from jax.experimental import pallas as pl


def kernel(x, edge_index, batch, W1, b1, W2, b2, fc1_W, fc1_b, fc2_W, fc2_b):
    raise NotImplementedError("write your pallas kernel here")


# speedup vs baseline: 15.5793x; 15.5793x over previous
import functools

import jax
import jax.numpy as jnp
from jax import lax
from jax.experimental import pallas as pl
from jax.experimental.pallas import tpu as pltpu
from jax.experimental.pallas import tpu_sc as plsc

N = 10000
E = 320000
F = 128
G = 64
N_PAD = 10240
E_PAD = 327680
N_CHUNKS = E_PAD // 128
CPT = N_CHUNKS // 16
CPW = N_CHUNKS // 32
RPT = N_PAD // 16
NB = N_PAD // 512
HALF = F // 2

_mesh = plsc.VectorSubcoreMesh(core_axis_name="c", subcore_axis_name="s")


@functools.partial(
    pl.kernel,
    out_type=jax.ShapeDtypeStruct((2, N_PAD, 16), jnp.float32),
    mesh=_mesh,
    scratch_types=[
        pltpu.VMEM_SHARED((N_PAD, 16), jnp.float32),
        pltpu.VMEM((CPW, 128), jnp.int32),
        pltpu.VMEM((128, 16), jnp.float32),
    ],
)
def _sc_degree(dst_hbm, zeros_hbm, deg_out, deg_s, dstv, ones_v):
    c = lax.axis_index("c")
    s = lax.axis_index("s")
    w = c * 16 + s
    pltpu.sync_copy(zeros_hbm.at[pl.ds(s * RPT, RPT)],
                    deg_s.at[pl.ds(s * RPT, RPT)])
    pltpu.sync_copy(dst_hbm.at[pl.ds(w * CPW, CPW)], dstv)

    def fill(i, carry):
        ones_v[i, :] = jnp.ones((16,), jnp.float32)
        return carry
    lax.fori_loop(0, 128, fill, 0)
    plsc.subcore_barrier()

    def body(j, carry):
        pltpu.sync_copy(ones_v, deg_s.at[dstv.at[j]], add=True)
        return carry
    lax.fori_loop(0, CPW, body, 0)
    plsc.subcore_barrier()
    pltpu.sync_copy(deg_s.at[pl.ds(s * RPT, RPT)],
                    deg_out.at[c, pl.ds(s * RPT, RPT)])


@functools.partial(
    pl.kernel,
    out_type=jax.ShapeDtypeStruct((2, N_PAD, HALF), jnp.float32),
    mesh=_mesh,
    scratch_types=[
        pltpu.VMEM_SHARED((N_PAD, HALF), jnp.float32),
        pltpu.VMEM_SHARED((N_PAD, HALF), jnp.float32),
        pltpu.VMEM((16, 128), jnp.int32),
        pltpu.VMEM((16, 128), jnp.int32),
        pltpu.VMEM((128, HALF), jnp.float32),
    ],
)
def _sc_spmm(u_hbm, init_hbm, src_hbm, dst_hbm, agg_out,
             u_s, acc_s, srcv, dstv, rows):
    c = lax.axis_index("c")
    s = lax.axis_index("s")
    pltpu.sync_copy(u_hbm.at[c, pl.ds(s * RPT, RPT)],
                    u_s.at[pl.ds(s * RPT, RPT)])
    pltpu.sync_copy(init_hbm.at[c, pl.ds(s * RPT, RPT)],
                    acc_s.at[pl.ds(s * RPT, RPT)])
    plsc.subcore_barrier()

    def outer(ob, carry):
        base = s * CPT + ob * 16
        pltpu.sync_copy(src_hbm.at[pl.ds(base, 16)], srcv)
        pltpu.sync_copy(dst_hbm.at[pl.ds(base, 16)], dstv)

        def body(j, c2):
            pltpu.sync_copy(u_s.at[srcv.at[j]], rows)
            pltpu.sync_copy(rows, acc_s.at[dstv.at[j]], add=True)
            return c2
        return lax.fori_loop(0, 16, body, carry)
    lax.fori_loop(0, CPT // 16, outer, 0)
    plsc.subcore_barrier()
    pltpu.sync_copy(acc_s.at[pl.ds(s * RPT, RPT)],
                    agg_out.at[c, pl.ds(s * RPT, RPT)])


def _tc_pre_body(parts_ref, x_ref, w1_ref, u_ref, init_ref, dinv_ref):
    deg = parts_ref[0, :, 0:1] + parts_ref[1, :, 0:1] + 1.0
    dinv = 1.0 / jnp.sqrt(deg)
    h = jnp.dot(x_ref[...], w1_ref[0], preferred_element_type=jnp.float32)
    u = h * dinv
    u_ref[...] = u[None]
    init_ref[...] = (u * dinv)[None]
    dinv_ref[...] = dinv


def _tc_pre(parts, x_pad, W1):
    return pl.pallas_call(
        _tc_pre_body,
        grid=(NB, 2),
        in_specs=[
            pl.BlockSpec((2, 512, 16), lambda g, c: (0, g, 0)),
            pl.BlockSpec((512, F), lambda g, c: (g, 0)),
            pl.BlockSpec((1, F, HALF), lambda g, c: (c, 0, 0)),
        ],
        out_specs=[
            pl.BlockSpec((1, 512, HALF), lambda g, c: (c, g, 0)),
            pl.BlockSpec((1, 512, HALF), lambda g, c: (c, g, 0)),
            pl.BlockSpec((512, 1), lambda g, c: (g, 0)),
        ],
        out_shape=[
            jax.ShapeDtypeStruct((2, N_PAD, HALF), jnp.float32),
            jax.ShapeDtypeStruct((2, N_PAD, HALF), jnp.float32),
            jax.ShapeDtypeStruct((N_PAD, 1), jnp.float32),
        ],
    )(parts, x_pad, W1)


def _tc_mid_body(agg_ref, dinv_ref, b1_ref, w2_ref, u2_ref, init2_ref):
    dinv = dinv_ref[...]
    out1 = jnp.concatenate([agg_ref[0], agg_ref[1]], axis=1)
    out1 = jnp.maximum(out1 * dinv + b1_ref[...][None, :], 0.0)
    h2 = jnp.dot(out1, w2_ref[0], preferred_element_type=jnp.float32)
    u2 = h2 * dinv
    u2_ref[...] = u2[None]
    init2_ref[...] = (u2 * dinv)[None]


def _tc_mid(agg1, dinv, b1, W2):
    return pl.pallas_call(
        _tc_mid_body,
        grid=(NB, 2),
        in_specs=[
            pl.BlockSpec((2, 512, HALF), lambda g, c: (0, g, 0)),
            pl.BlockSpec((512, 1), lambda g, c: (g, 0)),
            pl.BlockSpec((F,), lambda g, c: (0,)),
            pl.BlockSpec((1, F, HALF), lambda g, c: (c, 0, 0)),
        ],
        out_specs=[
            pl.BlockSpec((1, 512, HALF), lambda g, c: (c, g, 0)),
            pl.BlockSpec((1, 512, HALF), lambda g, c: (c, g, 0)),
        ],
        out_shape=[
            jax.ShapeDtypeStruct((2, N_PAD, HALF), jnp.float32),
            jax.ShapeDtypeStruct((2, N_PAD, HALF), jnp.float32),
        ],
    )(agg1, dinv, b1, W2)


def _tc_post_body(agg_ref, dinv_ref, b2_ref, batch_r_ref, batch_c_ref,
                  fc1w_ref, fc1b_ref, fc2w_ref, fc2b_ref, out_ref,
                  sum_acc, cnt_acc, max_acc):
    g = pl.program_id(0)

    @pl.when(g == 0)
    def _init():
        sum_acc[...] = jnp.zeros_like(sum_acc)
        cnt_acc[...] = jnp.zeros_like(cnt_acc)
        max_acc[...] = jnp.full_like(max_acc, -jnp.inf)

    dinv = dinv_ref[...]
    h = jnp.concatenate([agg_ref[0], agg_ref[1]], axis=1)
    h = jnp.maximum(h * dinv + b2_ref[...][None, :], 0.0)
    b_row = batch_r_ref[0]
    b_col = batch_c_ref[...]
    onehot = (b_row == lax.broadcasted_iota(jnp.int32, (G, 512), 0))
    onehot = onehot.astype(jnp.float32)
    sum_acc[...] += jnp.dot(onehot, h, preferred_element_type=jnp.float32)
    cnt_acc[...] += jnp.broadcast_to(
        jnp.sum(onehot, axis=1, keepdims=True), (G, F))

    lo = jnp.min(b_col)
    hi = jnp.minimum(jnp.max(b_col), G - 1)

    def mbody(gg, carry):
        mask = (b_col == gg)
        col = jnp.max(jnp.where(mask, h, -jnp.inf), axis=0, keepdims=True)
        max_acc[pl.ds(gg, 1), :] = jnp.maximum(max_acc[pl.ds(gg, 1), :], col)
        return carry
    lax.fori_loop(lo, hi + 1, mbody, 0)

    @pl.when(g == NB - 1)
    def _fin():
        mean = sum_acc[...] / jnp.maximum(cnt_acc[...], 1.0)
        mx = max_acc[...]
        mx = jnp.where(mx == -jnp.inf, 0.0, mx)
        z = (jnp.dot(mean, fc1w_ref[0:F, :], preferred_element_type=jnp.float32)
             + jnp.dot(mx, fc1w_ref[F:2 * F, :],
                       preferred_element_type=jnp.float32)
             + fc1b_ref[...][None, :])
        z = jnp.maximum(z, 0.0)
        logits = jnp.dot(z, fc2w_ref[...],
                         preferred_element_type=jnp.float32)
        logits = logits + fc2b_ref[...][None, :]
        m = jnp.max(logits, axis=1, keepdims=True)
        lse = jnp.log(jnp.sum(jnp.exp(logits - m), axis=1, keepdims=True)) + m
        out_ref[...] = logits - lse


def _tc_post(agg2, dinv, b2, batch_r, batch_c, fc1_W, fc1_b, fc2_W, fc2_b):
    return pl.pallas_call(
        _tc_post_body,
        grid=(NB,),
        in_specs=[
            pl.BlockSpec((2, 512, HALF), lambda g: (0, g, 0)),
            pl.BlockSpec((512, 1), lambda g: (g, 0)),
            pl.BlockSpec((F,), lambda g: (0,)),
            pl.BlockSpec((1, 1, 512), lambda g: (g, 0, 0)),
            pl.BlockSpec((512, 1), lambda g: (g, 0)),
            pl.BlockSpec((2 * F, F), lambda g: (0, 0)),
            pl.BlockSpec((F,), lambda g: (0,)),
            pl.BlockSpec((F, 10), lambda g: (0, 0)),
            pl.BlockSpec((10,), lambda g: (0,)),
        ],
        out_specs=pl.BlockSpec((G, 10), lambda g: (0, 0)),
        out_shape=jax.ShapeDtypeStruct((G, 10), jnp.float32),
        scratch_shapes=[
            pltpu.VMEM((G, F), jnp.float32),
            pltpu.VMEM((G, F), jnp.float32),
            pltpu.VMEM((G, F), jnp.float32),
        ],
    )(agg2, dinv, b2, batch_r, batch_c, fc1_W, fc1_b, fc2_W, fc2_b)


def kernel(x, edge_index, batch, W1, b1, W2, b2, fc1_W, fc1_b, fc2_W, fc2_b):
    ei = edge_index.astype(jnp.int32)
    pad_e = E_PAD - E
    src = jnp.concatenate([ei[0], jnp.zeros((pad_e,), jnp.int32)])
    dst = jnp.concatenate([ei[1], jnp.full((pad_e,), N, jnp.int32)])
    src2d = src.reshape(N_CHUNKS, 128)
    dst2d = dst.reshape(N_CHUNKS, 128)
    x_pad = jnp.pad(x, ((0, N_PAD - N), (0, 0)))
    batch_pad = jnp.concatenate(
        [batch.astype(jnp.int32), jnp.full((N_PAD - N,), G, jnp.int32)])
    batch_r = batch_pad.reshape(NB, 1, 512)
    batch_c = batch_pad.reshape(N_PAD, 1)
    zeros = jnp.zeros((N_PAD, 16), jnp.float32)

    W1r = W1.reshape(F, 2, HALF).transpose(1, 0, 2)
    W2r = W2.reshape(F, 2, HALF).transpose(1, 0, 2)

    parts = _sc_degree(dst2d, zeros)
    u1, init1, dinv = _tc_pre(parts, x_pad, W1r)
    agg1 = _sc_spmm(u1, init1, src2d, dst2d)
    u2, init2 = _tc_mid(agg1, dinv, b1, W2r)
    agg2 = _sc_spmm(u2, init2, src2d, dst2d)
    return _tc_post(agg2, dinv, b2, batch_r, batch_c,
                    fc1_W, fc1_b, fc2_W, fc2_b)
